# baseline (speedup 1.0000x reference)
import functools

import jax
import jax.numpy as jnp
from jax import lax
from jax.experimental import pallas as pl
from jax.experimental.pallas import tpu as pltpu
from jax.experimental.pallas import tpu_sc as plsc

N = 10000
D = 256
H = 128
E = 160000
NC = 2
NS = 16
L = 16

NPAD = 10240
CHN = NPAD // NS
EPT = E // NS
CKE = 128
NCHK = 79
NB = 2
PKT = NCHK * CKE
RPT = NPAD // NS

_MESH = plsc.VectorSubcoreMesh(core_axis_name="c", subcore_axis_name="s")
_SC_PARAMS = pltpu.CompilerParams(needs_layout_passes=False)


def _rsqrt16(d):
    i = plsc.bitcast(d, jnp.int32)
    i = jnp.int32(0x5F3759DF) - lax.shift_right_logical(i, jnp.int32(1))
    y = plsc.bitcast(i, jnp.float32)
    for _ in range(3):
        y = y * (1.5 - 0.5 * d * y * y)
    return y


def _dinv_body(dst_hbm, dinv_hbm, dst_v, deg_v, part_s, red_v, tmp_v):
    c = lax.axis_index("c")
    s = lax.axis_index("s")

    zero16 = jnp.zeros((L,), jnp.float32)

    def _zero(i, carry):
        deg_v[pl.ds(i * L, L)] = zero16
        return carry

    lax.fori_loop(0, NPAD // L, _zero, 0)

    pltpu.sync_copy(dst_hbm.at[pl.ds(s * EPT, EPT)], dst_v)

    ones16 = jnp.ones((L,), jnp.float32)

    def _count(i, carry):
        idx = dst_v[pl.ds(i * L, L)]
        plsc.addupdate_scatter(deg_v, [idx], ones16)
        return carry

    lax.fori_loop(0, EPT // L, _count, 0)

    pltpu.sync_copy(deg_v, part_s.at[s])
    plsc.subcore_barrier()

    col = s * CHN
    pltpu.sync_copy(part_s.at[0, pl.ds(col, CHN)], red_v)

    def _accum(j, carry):
        pltpu.sync_copy(part_s.at[j, pl.ds(col, CHN)], tmp_v)

        def _vadd(k, c2):
            red_v[pl.ds(k * L, L)] = red_v[pl.ds(k * L, L)] + tmp_v[pl.ds(k * L, L)]
            return c2

        lax.fori_loop(0, CHN // L, _vadd, 0)
        return carry

    lax.fori_loop(1, NS, _accum, 0)

    def _rs(k, carry):
        d = red_v[pl.ds(k * L, L)] + 1.0
        red_v[pl.ds(k * L, L)] = _rsqrt16(d)
        return carry

    lax.fori_loop(0, CHN // L, _rs, 0)

    @pl.when(c == 0)
    def _():
        pltpu.sync_copy(red_v, dinv_hbm.at[pl.ds(col, CHN)])


@functools.partial(
    pl.kernel,
    out_type=jax.ShapeDtypeStruct((NPAD,), jnp.float32),
    mesh=_MESH,
    compiler_params=_SC_PARAMS,
    scratch_types=[
        pltpu.VMEM((EPT,), jnp.int32),
        pltpu.VMEM((NPAD,), jnp.float32),
        pltpu.VMEM_SHARED((NS, NPAD), jnp.float32),
        pltpu.VMEM((CHN,), jnp.float32),
        pltpu.VMEM((CHN,), jnp.float32),
    ],
)
def _dinv_kernel(dst_hbm, dinv_hbm, dst_v, deg_v, part_s, red_v, tmp_v):
    _dinv_body(dst_hbm, dinv_hbm, dst_v, deg_v, part_s, red_v, tmp_v)


def _agg_body(g_hbm, pk_hbm, t_hbm, acc_s, pk_v, sidx_v, didx_v, rows_v,
              gsem, ssem):
    c = lax.axis_index("c")
    s = lax.axis_index("s")

    r0 = s * RPT
    pltpu.sync_copy(g_hbm.at[pl.ds(c * NPAD + r0, RPT)], acc_s.at[pl.ds(r0, RPT)])

    pltpu.sync_copy(pk_hbm.at[s], pk_v)
    plsc.subcore_barrier()

    coff = c * NPAD
    mask = jnp.int32(0xFFFF)

    def _unpack(j, p):
        base = j * CKE

        def _u(k, carry):
            v = pk_v[pl.ds(base + k * L, L)]
            sidx_v[p, pl.ds(k * L, L)] = (
                lax.shift_right_logical(v, jnp.int32(16)) + coff)
            didx_v[p, pl.ds(k * L, L)] = lax.bitwise_and(v, mask)
            return carry

        lax.fori_loop(0, CKE // L, _u, 0)

    def _gather(p):
        return pltpu.make_async_copy(
            g_hbm.at[sidx_v.at[p]], rows_v.at[p], gsem)

    def _scatter(p):
        return pltpu.make_async_copy(
            rows_v.at[p], acc_s.at[didx_v.at[p]], ssem)

    _unpack(0, 0)
    _gather(0).start()
    _unpack(1, 1)
    _gather(1).start()
    _gather(0).wait()
    _scatter(0).start(add=True)
    _gather(1).wait()
    _scatter(1).start(add=True)

    def _step(j, carry):
        p = lax.rem(j, NB)
        _scatter(p).wait()
        _unpack(j, p)
        _gather(p).start()
        _gather(p).wait()
        _scatter(p).start(add=True)
        return carry

    lax.fori_loop(NB, NCHK, _step, 0)

    for _ in range(NB):
        _scatter(0).wait()

    plsc.subcore_barrier()
    pltpu.sync_copy(acc_s.at[pl.ds(r0, RPT)], t_hbm.at[pl.ds(c * NPAD + r0, RPT)])


def _make_agg(dtype):
    @functools.partial(
        pl.kernel,
        out_type=jax.ShapeDtypeStruct((NC * NPAD, H), dtype),
        mesh=_MESH,
        compiler_params=_SC_PARAMS,
        scratch_types=[
            pltpu.VMEM_SHARED((NPAD, H), dtype),
            pltpu.VMEM((PKT,), jnp.int32),
            pltpu.VMEM((NB, CKE), jnp.int32),
            pltpu.VMEM((NB, CKE), jnp.int32),
            pltpu.VMEM((NB, CKE, H), dtype),
            pltpu.SemaphoreType.DMA,
            pltpu.SemaphoreType.DMA,
        ],
    )
    def _k(g_hbm, pk_hbm, t_hbm, acc_s, pk_v, sidx_v, didx_v, rows_v,
           gsem, ssem):
        _agg_body(g_hbm, pk_hbm, t_hbm, acc_s, pk_v, sidx_v, didx_v, rows_v,
                  gsem, ssem)

    return _k


_agg_kernel = _make_agg(jnp.float32)


BR = 2000
NBLK = N // BR


def _mm1_body(x_ref, w_ref, dinv_ref, out_ref):
    h = jnp.dot(x_ref[...], w_ref[...], preferred_element_type=jnp.float32)
    g = h * dinv_ref[...]
    out_ref[0] = g[:, :H]
    out_ref[1] = g[:, H:]


def _mm1_call(x, W1, dinv):
    return pl.pallas_call(
        _mm1_body,
        grid=(NBLK,),
        in_specs=[
            pl.BlockSpec((BR, D), lambda i: (i, 0)),
            pl.BlockSpec((D, D), lambda i: (0, 0)),
            pl.BlockSpec((BR, 1), lambda i: (i, 0)),
        ],
        out_specs=pl.BlockSpec((NC, BR, H), lambda i: (0, i, 0)),
        out_shape=jax.ShapeDtypeStruct((NC, NPAD, H), jnp.float32),
    )(x, W1, dinv)


def _mm2_body(t_ref, dinv_ref, b1_ref, w_ref, out_ref):
    d = dinv_ref[...]
    z0 = t_ref[0] * d + b1_ref[0:1, :H]
    z1 = t_ref[1] * d + b1_ref[0:1, H:]
    z0 = jnp.where(z0 >= 0, z0, z0 * 0.01)
    z1 = jnp.where(z1 >= 0, z1, z1 * 0.01)
    h = jnp.dot(z0, w_ref[:H, :], preferred_element_type=jnp.float32)
    h = h + jnp.dot(z1, w_ref[H:, :], preferred_element_type=jnp.float32)
    g = h * d
    out_ref[0] = g[:, :H]
    out_ref[1] = g[:, H:]


def _mm2_call(t1, dinv, b1, W2):
    return pl.pallas_call(
        _mm2_body,
        grid=(NBLK,),
        in_specs=[
            pl.BlockSpec((NC, BR, H), lambda i: (0, i, 0)),
            pl.BlockSpec((BR, 1), lambda i: (i, 0)),
            pl.BlockSpec((1, D), lambda i: (0, 0)),
            pl.BlockSpec((D, D), lambda i: (0, 0)),
        ],
        out_specs=pl.BlockSpec((NC, BR, H), lambda i: (0, i, 0)),
        out_shape=jax.ShapeDtypeStruct((NC, NPAD, H), jnp.float32),
    )(t1, dinv, b1, W2)


def _final_body(t_ref, dinv_ref, b2_ref, out_ref):
    d = dinv_ref[...]
    out_ref[:, :H] = t_ref[0] * d + b2_ref[0:1, :H]
    out_ref[:, H:] = t_ref[1] * d + b2_ref[0:1, H:]


def _final_call(t2, dinv, b2):
    return pl.pallas_call(
        _final_body,
        grid=(NBLK,),
        in_specs=[
            pl.BlockSpec((NC, BR, H), lambda i: (0, i, 0)),
            pl.BlockSpec((BR, 1), lambda i: (i, 0)),
            pl.BlockSpec((1, D), lambda i: (0, 0)),
        ],
        out_specs=pl.BlockSpec((BR, D), lambda i: (i, 0)),
        out_shape=jax.ShapeDtypeStruct((N, D), jnp.float32),
    )(t2, dinv, b2)


def kernel(x, edge_index, W1, b1, W2, b2):
    ei = edge_index.astype(jnp.int32)
    src = ei[0]
    dst = ei[1]
    packed = jnp.sort(src * 65536 + dst)
    packed = jnp.pad(packed.reshape(NS, EPT), ((0, 0), (0, PKT - EPT)),
                     constant_values=N)

    dinv_pad = _dinv_kernel(dst)
    dinv = dinv_pad[:N].reshape(N, 1)

    g1 = _mm1_call(x, W1, dinv)
    t1 = _agg_kernel(g1.reshape(NC * NPAD, H), packed)
    g2 = _mm2_call(t1.reshape(NC, NPAD, H), dinv, b1.reshape(1, D), W2)
    t2 = _agg_kernel(g2.reshape(NC * NPAD, H), packed)
    return _final_call(t2.reshape(NC, NPAD, H), dinv, b2.reshape(1, D))


# speedup vs baseline: 1.7767x; 1.7767x over previous
import functools

import jax
import jax.numpy as jnp
from jax import lax
from jax.experimental import pallas as pl
from jax.experimental.pallas import tpu as pltpu
from jax.experimental.pallas import tpu_sc as plsc

N = 10000
D = 256
H = 128
E = 160000
NC = 2
NS = 16
L = 16

NPAD = 10240
CHN = NPAD // NS
EPT = E // NS
CKE = 128
NCHK = 79
NB = 2
PKT = NCHK * CKE
RPT = NPAD // NS

_MESH = plsc.VectorSubcoreMesh(core_axis_name="c", subcore_axis_name="s")
_SC_PARAMS = pltpu.CompilerParams(needs_layout_passes=False)


def _rsqrt16(d):
    i = plsc.bitcast(d, jnp.int32)
    i = jnp.int32(0x5F3759DF) - lax.shift_right_logical(i, jnp.int32(1))
    y = plsc.bitcast(i, jnp.float32)
    for _ in range(3):
        y = y * (1.5 - 0.5 * d * y * y)
    return y


def _dinv_body(dst_hbm, dinv_hbm, dst_v, deg_v, part_s, red_v, tmp_v):
    c = lax.axis_index("c")
    s = lax.axis_index("s")

    zero16 = jnp.zeros((L,), jnp.float32)

    def _zero(i, carry):
        deg_v[pl.ds(i * L, L)] = zero16
        return carry

    lax.fori_loop(0, NPAD // L, _zero, 0)

    pltpu.sync_copy(dst_hbm.at[pl.ds(s * EPT, EPT)], dst_v)

    ones16 = jnp.ones((L,), jnp.float32)

    def _count(i, carry):
        idx = dst_v[pl.ds(i * L, L)]
        plsc.addupdate_scatter(deg_v, [idx], ones16)
        return carry

    lax.fori_loop(0, EPT // L, _count, 0)

    pltpu.sync_copy(deg_v, part_s.at[s])
    plsc.subcore_barrier()

    col = s * CHN
    pltpu.sync_copy(part_s.at[0, pl.ds(col, CHN)], red_v)

    def _accum(j, carry):
        pltpu.sync_copy(part_s.at[j, pl.ds(col, CHN)], tmp_v)

        def _vadd(k, c2):
            red_v[pl.ds(k * L, L)] = red_v[pl.ds(k * L, L)] + tmp_v[pl.ds(k * L, L)]
            return c2

        lax.fori_loop(0, CHN // L, _vadd, 0)
        return carry

    lax.fori_loop(1, NS, _accum, 0)

    def _rs(k, carry):
        d = red_v[pl.ds(k * L, L)] + 1.0
        red_v[pl.ds(k * L, L)] = _rsqrt16(d)
        return carry

    lax.fori_loop(0, CHN // L, _rs, 0)

    @pl.when(c == 0)
    def _():
        pltpu.sync_copy(red_v, dinv_hbm.at[pl.ds(col, CHN)])


@functools.partial(
    pl.kernel,
    out_type=jax.ShapeDtypeStruct((NPAD,), jnp.float32),
    mesh=_MESH,
    compiler_params=_SC_PARAMS,
    scratch_types=[
        pltpu.VMEM((EPT,), jnp.int32),
        pltpu.VMEM((NPAD,), jnp.float32),
        pltpu.VMEM_SHARED((NS, NPAD), jnp.float32),
        pltpu.VMEM((CHN,), jnp.float32),
        pltpu.VMEM((CHN,), jnp.float32),
    ],
)
def _dinv_kernel(dst_hbm, dinv_hbm, dst_v, deg_v, part_s, red_v, tmp_v):
    _dinv_body(dst_hbm, dinv_hbm, dst_v, deg_v, part_s, red_v, tmp_v)


def _agg_body(g_hbm, pk_hbm, t_hbm, acc_s, pk_v, sidx_v, didx_v, rows_v,
              gsem, ssem):
    c = lax.axis_index("c")
    s = lax.axis_index("s")

    r0 = s * RPT
    pltpu.sync_copy(g_hbm.at[pl.ds(c * NPAD + r0, RPT)], acc_s.at[pl.ds(r0, RPT)])

    pltpu.sync_copy(pk_hbm.at[s], pk_v)
    plsc.subcore_barrier()

    coff = c * NPAD
    mask = jnp.int32(0xFFFF)

    def _unpack(j, p):
        base = j * CKE

        def _u(k, carry):
            v = pk_v[pl.ds(base + k * L, L)]
            sidx_v[p, pl.ds(k * L, L)] = lax.bitwise_and(v, mask) + coff
            didx_v[p, pl.ds(k * L, L)] = lax.shift_right_logical(v, jnp.int32(16))
            return carry

        lax.fori_loop(0, CKE // L, _u, 0)

    def _gather(p):
        return pltpu.make_async_copy(
            g_hbm.at[sidx_v.at[p]], rows_v.at[p], gsem)

    def _scatter(p):
        return pltpu.make_async_copy(
            rows_v.at[p], acc_s.at[didx_v.at[p]], ssem)

    _unpack(0, 0)
    _gather(0).start()
    _unpack(1, 1)
    _gather(1).start()
    _gather(0).wait()
    _scatter(0).start(add=True)
    _gather(1).wait()
    _scatter(1).start(add=True)

    def _step(j, carry):
        p = lax.rem(j, NB)
        _scatter(p).wait()
        _unpack(j, p)
        _gather(p).start()
        _gather(p).wait()
        _scatter(p).start(add=True)
        return carry

    lax.fori_loop(NB, NCHK, _step, 0)

    for _ in range(NB):
        _scatter(0).wait()

    plsc.subcore_barrier()
    pltpu.sync_copy(acc_s.at[pl.ds(r0, RPT)], t_hbm.at[pl.ds(c * NPAD + r0, RPT)])


def _make_agg(dtype):
    @functools.partial(
        pl.kernel,
        out_type=jax.ShapeDtypeStruct((NC * NPAD, H), dtype),
        mesh=_MESH,
        compiler_params=_SC_PARAMS,
        scratch_types=[
            pltpu.VMEM_SHARED((NPAD, H), dtype),
            pltpu.VMEM((PKT,), jnp.int32),
            pltpu.VMEM((NB, CKE), jnp.int32),
            pltpu.VMEM((NB, CKE), jnp.int32),
            pltpu.VMEM((NB, CKE, H), dtype),
            pltpu.SemaphoreType.DMA,
            pltpu.SemaphoreType.DMA,
        ],
    )
    def _k(g_hbm, pk_hbm, t_hbm, acc_s, pk_v, sidx_v, didx_v, rows_v,
           gsem, ssem):
        _agg_body(g_hbm, pk_hbm, t_hbm, acc_s, pk_v, sidx_v, didx_v, rows_v,
                  gsem, ssem)

    return _k


_agg_kernel = _make_agg(jnp.float32)


BR = 2000
NBLK = N // BR


def _mm1_body(x_ref, w_ref, dinv_ref, out_ref):
    h = jnp.dot(x_ref[...], w_ref[...], preferred_element_type=jnp.float32)
    g = h * dinv_ref[...]
    out_ref[0] = g[:, :H]
    out_ref[1] = g[:, H:]


def _mm1_call(x, W1, dinv):
    return pl.pallas_call(
        _mm1_body,
        grid=(NBLK,),
        in_specs=[
            pl.BlockSpec((BR, D), lambda i: (i, 0)),
            pl.BlockSpec((D, D), lambda i: (0, 0)),
            pl.BlockSpec((BR, 1), lambda i: (i, 0)),
        ],
        out_specs=pl.BlockSpec((NC, BR, H), lambda i: (0, i, 0)),
        out_shape=jax.ShapeDtypeStruct((NC, NPAD, H), jnp.float32),
    )(x, W1, dinv)


def _mm2_body(t_ref, dinv_ref, b1_ref, w_ref, out_ref):
    d = dinv_ref[...]
    z0 = t_ref[0] * d + b1_ref[0:1, :H]
    z1 = t_ref[1] * d + b1_ref[0:1, H:]
    z0 = jnp.where(z0 >= 0, z0, z0 * 0.01)
    z1 = jnp.where(z1 >= 0, z1, z1 * 0.01)
    h = jnp.dot(z0, w_ref[:H, :], preferred_element_type=jnp.float32)
    h = h + jnp.dot(z1, w_ref[H:, :], preferred_element_type=jnp.float32)
    g = h * d
    out_ref[0] = g[:, :H]
    out_ref[1] = g[:, H:]


def _mm2_call(t1, dinv, b1, W2):
    return pl.pallas_call(
        _mm2_body,
        grid=(NBLK,),
        in_specs=[
            pl.BlockSpec((NC, BR, H), lambda i: (0, i, 0)),
            pl.BlockSpec((BR, 1), lambda i: (i, 0)),
            pl.BlockSpec((1, D), lambda i: (0, 0)),
            pl.BlockSpec((D, D), lambda i: (0, 0)),
        ],
        out_specs=pl.BlockSpec((NC, BR, H), lambda i: (0, i, 0)),
        out_shape=jax.ShapeDtypeStruct((NC, NPAD, H), jnp.float32),
    )(t1, dinv, b1, W2)


def _final_body(t_ref, dinv_ref, b2_ref, out_ref):
    d = dinv_ref[...]
    out_ref[:, :H] = t_ref[0] * d + b2_ref[0:1, :H]
    out_ref[:, H:] = t_ref[1] * d + b2_ref[0:1, H:]


def _final_call(t2, dinv, b2):
    return pl.pallas_call(
        _final_body,
        grid=(NBLK,),
        in_specs=[
            pl.BlockSpec((NC, BR, H), lambda i: (0, i, 0)),
            pl.BlockSpec((BR, 1), lambda i: (i, 0)),
            pl.BlockSpec((1, D), lambda i: (0, 0)),
        ],
        out_specs=pl.BlockSpec((BR, D), lambda i: (i, 0)),
        out_shape=jax.ShapeDtypeStruct((N, D), jnp.float32),
    )(t2, dinv, b2)


def kernel(x, edge_index, W1, b1, W2, b2):
    ei = edge_index.astype(jnp.int32)
    src = ei[0]
    dst = ei[1]
    packed = dst * 65536 + src
    packed = jnp.pad(packed.reshape(NS, EPT), ((0, 0), (0, PKT - EPT)),
                     constant_values=N * 65536)

    dinv_pad = _dinv_kernel(dst)
    dinv = dinv_pad[:N].reshape(N, 1)

    g1 = _mm1_call(x, W1, dinv)
    t1 = _agg_kernel(g1.reshape(NC * NPAD, H), packed)
    g2 = _mm2_call(t1.reshape(NC, NPAD, H), dinv, b1.reshape(1, D), W2)
    t2 = _agg_kernel(g2.reshape(NC * NPAD, H), packed)
    return _final_call(t2.reshape(NC, NPAD, H), dinv, b2.reshape(1, D))


# speedup vs baseline: 2.0999x; 1.1819x over previous
import functools

import jax
import jax.numpy as jnp
from jax import lax
from jax.experimental import pallas as pl
from jax.experimental.pallas import tpu as pltpu
from jax.experimental.pallas import tpu_sc as plsc

N = 10000
D = 256
H = 128
E = 160000
NC = 2
NS = 16
L = 16

NPAD = 10240
CHN = NPAD // NS
EPT = E // NS
CKE = 80
NCHK = 125
NB = 2
PKT = NCHK * CKE
RPT = NPAD // NS

_MESH = plsc.VectorSubcoreMesh(core_axis_name="c", subcore_axis_name="s")
_SC_PARAMS = pltpu.CompilerParams(needs_layout_passes=False)


def _rsqrt16(d):
    i = plsc.bitcast(d, jnp.int32)
    i = jnp.int32(0x5F3759DF) - lax.shift_right_logical(i, jnp.int32(1))
    y = plsc.bitcast(i, jnp.float32)
    for _ in range(3):
        y = y * (1.5 - 0.5 * d * y * y)
    return y


def _dinv_body(dst_hbm, dinv_hbm, dst_v, deg_v, part_s, red_v, tmp_v):
    c = lax.axis_index("c")
    s = lax.axis_index("s")

    zero16 = jnp.zeros((L,), jnp.float32)

    def _zero(i, carry):
        deg_v[pl.ds(i * L, L)] = zero16
        return carry

    lax.fori_loop(0, NPAD // L, _zero, 0)

    pltpu.sync_copy(dst_hbm.at[pl.ds(s * EPT, EPT)], dst_v)

    ones16 = jnp.ones((L,), jnp.float32)

    def _count(i, carry):
        idx = dst_v[pl.ds(i * L, L)]
        plsc.addupdate_scatter(deg_v, [idx], ones16)
        return carry

    lax.fori_loop(0, EPT // L, _count, 0)

    pltpu.sync_copy(deg_v, part_s.at[s])
    plsc.subcore_barrier()

    col = s * CHN
    pltpu.sync_copy(part_s.at[0, pl.ds(col, CHN)], red_v)

    def _accum(j, carry):
        pltpu.sync_copy(part_s.at[j, pl.ds(col, CHN)], tmp_v)

        def _vadd(k, c2):
            red_v[pl.ds(k * L, L)] = red_v[pl.ds(k * L, L)] + tmp_v[pl.ds(k * L, L)]
            return c2

        lax.fori_loop(0, CHN // L, _vadd, 0)
        return carry

    lax.fori_loop(1, NS, _accum, 0)

    def _rs(k, carry):
        d = red_v[pl.ds(k * L, L)] + 1.0
        red_v[pl.ds(k * L, L)] = _rsqrt16(d)
        return carry

    lax.fori_loop(0, CHN // L, _rs, 0)

    @pl.when(c == 0)
    def _():
        pltpu.sync_copy(red_v, dinv_hbm.at[pl.ds(col, CHN)])


@functools.partial(
    pl.kernel,
    out_type=jax.ShapeDtypeStruct((NPAD,), jnp.float32),
    mesh=_MESH,
    compiler_params=_SC_PARAMS,
    scratch_types=[
        pltpu.VMEM((EPT,), jnp.int32),
        pltpu.VMEM((NPAD,), jnp.float32),
        pltpu.VMEM_SHARED((NS, NPAD), jnp.float32),
        pltpu.VMEM((CHN,), jnp.float32),
        pltpu.VMEM((CHN,), jnp.float32),
    ],
)
def _dinv_kernel(dst_hbm, dinv_hbm, dst_v, deg_v, part_s, red_v, tmp_v):
    _dinv_body(dst_hbm, dinv_hbm, dst_v, deg_v, part_s, red_v, tmp_v)


def _agg_body(g_hbm, pk_hbm, t_hbm, acc_s, pk_v, sidx_v, didx_v, rows_v,
              gsem, ssem):
    c = lax.axis_index("c")
    s = lax.axis_index("s")

    r0 = s * RPT
    pltpu.sync_copy(g_hbm.at[pl.ds(c * NPAD + r0, RPT)], acc_s.at[pl.ds(r0, RPT)])

    pltpu.sync_copy(pk_hbm.at[s], pk_v)
    plsc.subcore_barrier()

    coff = c * NPAD
    mask = jnp.int32(0xFFFF)

    def _unpack(j, p):
        base = j * CKE

        def _u(k, carry):
            v = pk_v[pl.ds(base + k * L, L)]
            sidx_v[p, pl.ds(k * L, L)] = lax.bitwise_and(v, mask) + coff
            didx_v[p, pl.ds(k * L, L)] = lax.shift_right_logical(v, jnp.int32(16))
            return carry

        lax.fori_loop(0, CKE // L, _u, 0)

    def _gather(p):
        return pltpu.make_async_copy(
            g_hbm.at[sidx_v.at[p]], rows_v.at[p], gsem)

    def _scatter(p):
        return pltpu.make_async_copy(
            rows_v.at[p], acc_s.at[didx_v.at[p]], ssem)

    _unpack(0, 0)
    _gather(0).start()
    _unpack(1, 1)
    _gather(1).start()
    _gather(0).wait()
    _scatter(0).start(add=True)
    _gather(1).wait()
    _scatter(1).start(add=True)

    def _step(j, carry):
        p = lax.rem(j, NB)
        _scatter(p).wait()
        _unpack(j, p)
        _gather(p).start()
        _gather(p).wait()
        _scatter(p).start(add=True)
        return carry

    lax.fori_loop(NB, NCHK, _step, 0)

    for _ in range(NB):
        _scatter(0).wait()

    plsc.subcore_barrier()
    pltpu.sync_copy(acc_s.at[pl.ds(r0, RPT)], t_hbm.at[pl.ds(c * NPAD + r0, RPT)])


def _make_agg(dtype):
    @functools.partial(
        pl.kernel,
        out_type=jax.ShapeDtypeStruct((NC * NPAD, H), dtype),
        mesh=_MESH,
        compiler_params=_SC_PARAMS,
        scratch_types=[
            pltpu.VMEM_SHARED((NPAD, H), dtype),
            pltpu.VMEM((PKT,), jnp.int32),
            pltpu.VMEM((NB, CKE), jnp.int32),
            pltpu.VMEM((NB, CKE), jnp.int32),
            pltpu.VMEM((NB, CKE, H), dtype),
            pltpu.SemaphoreType.DMA,
            pltpu.SemaphoreType.DMA,
        ],
    )
    def _k(g_hbm, pk_hbm, t_hbm, acc_s, pk_v, sidx_v, didx_v, rows_v,
           gsem, ssem):
        _agg_body(g_hbm, pk_hbm, t_hbm, acc_s, pk_v, sidx_v, didx_v, rows_v,
                  gsem, ssem)

    return _k


_agg_kernel = _make_agg(jnp.float32)


BR = 2000
NBLK = N // BR


def _mm1_body(x_ref, w_ref, dinv_ref, out_ref):
    h = jnp.dot(x_ref[...], w_ref[...], preferred_element_type=jnp.float32)
    g = h * dinv_ref[...]
    out_ref[0] = g[:, :H]
    out_ref[1] = g[:, H:]


def _mm1_call(x, W1, dinv):
    return pl.pallas_call(
        _mm1_body,
        grid=(NBLK,),
        in_specs=[
            pl.BlockSpec((BR, D), lambda i: (i, 0)),
            pl.BlockSpec((D, D), lambda i: (0, 0)),
            pl.BlockSpec((BR, 1), lambda i: (i, 0)),
        ],
        out_specs=pl.BlockSpec((NC, BR, H), lambda i: (0, i, 0)),
        out_shape=jax.ShapeDtypeStruct((NC, NPAD, H), jnp.float32),
    )(x, W1, dinv)


def _mm2_body(t_ref, dinv_ref, b1_ref, w_ref, out_ref):
    d = dinv_ref[...]
    z0 = t_ref[0] * d + b1_ref[0:1, :H]
    z1 = t_ref[1] * d + b1_ref[0:1, H:]
    z0 = jnp.where(z0 >= 0, z0, z0 * 0.01)
    z1 = jnp.where(z1 >= 0, z1, z1 * 0.01)
    h = jnp.dot(z0, w_ref[:H, :], preferred_element_type=jnp.float32)
    h = h + jnp.dot(z1, w_ref[H:, :], preferred_element_type=jnp.float32)
    g = h * d
    out_ref[0] = g[:, :H]
    out_ref[1] = g[:, H:]


def _mm2_call(t1, dinv, b1, W2):
    return pl.pallas_call(
        _mm2_body,
        grid=(NBLK,),
        in_specs=[
            pl.BlockSpec((NC, BR, H), lambda i: (0, i, 0)),
            pl.BlockSpec((BR, 1), lambda i: (i, 0)),
            pl.BlockSpec((1, D), lambda i: (0, 0)),
            pl.BlockSpec((D, D), lambda i: (0, 0)),
        ],
        out_specs=pl.BlockSpec((NC, BR, H), lambda i: (0, i, 0)),
        out_shape=jax.ShapeDtypeStruct((NC, NPAD, H), jnp.float32),
    )(t1, dinv, b1, W2)


def _final_body(t_ref, dinv_ref, b2_ref, out_ref):
    d = dinv_ref[...]
    out_ref[:, :H] = t_ref[0] * d + b2_ref[0:1, :H]
    out_ref[:, H:] = t_ref[1] * d + b2_ref[0:1, H:]


def _final_call(t2, dinv, b2):
    return pl.pallas_call(
        _final_body,
        grid=(NBLK,),
        in_specs=[
            pl.BlockSpec((NC, BR, H), lambda i: (0, i, 0)),
            pl.BlockSpec((BR, 1), lambda i: (i, 0)),
            pl.BlockSpec((1, D), lambda i: (0, 0)),
        ],
        out_specs=pl.BlockSpec((BR, D), lambda i: (i, 0)),
        out_shape=jax.ShapeDtypeStruct((N, D), jnp.float32),
    )(t2, dinv, b2)


def kernel(x, edge_index, W1, b1, W2, b2):
    ei = edge_index.astype(jnp.int32)
    src = ei[0]
    dst = ei[1]
    packed = (dst * 65536 + src).reshape(NS, EPT)

    dinv_pad = _dinv_kernel(dst)
    dinv = dinv_pad[:N].reshape(N, 1)

    g1 = _mm1_call(x, W1, dinv)
    t1 = _agg_kernel(g1.reshape(NC * NPAD, H), packed)
    g2 = _mm2_call(t1.reshape(NC, NPAD, H), dinv, b1.reshape(1, D), W2)
    t2 = _agg_kernel(g2.reshape(NC * NPAD, H), packed)
    return _final_call(t2.reshape(NC, NPAD, H), dinv, b2.reshape(1, D))
